# Initial kernel scaffold; baseline (speedup 1.0000x reference)
import jax
import jax.numpy as jnp
from jax.experimental import pallas as pl


def kernel(x, g1a, c1b, g2a, c2b, g3c, c3b, k3):
    raise NotImplementedError("write your pallas kernel here")


# speedup vs baseline: 1.8327x; 1.8327x over previous
import jax
import jax.numpy as jnp
from jax.experimental import pallas as pl
from jax.experimental.pallas import tpu as pltpu

_INV_SQRT2 = 0.7071067811865476
_G = 8
_B = 4


def _gelu_erf(v):
    return (v * 0.5) * (jax.lax.erf(v * _INV_SQRT2) + 1.0)


def _fused_kernel(x_ref, c1h_ref, g1_ref, g2s_ref, c2x_ref, b2_ref,
                  bd_ref, c3v_ref, k3r_ref, o_ref,
                  t1_ref, u1_ref, a1_ref, y2_ref, yc_ref, a2_ref, p3_ref):
    f32 = jnp.float32
    bf = jnp.bfloat16
    G = _G

    t1_ref[...] = jnp.dot(x_ref[...], c1h_ref[...],
                          preferred_element_type=f32).astype(bf)
    for kw in range(3):
        for g in range(G):
            u1_ref[pl.ds(kw * 96, 96), pl.ds(g * 128, 128)] = \
                t1_ref[pl.ds(g * 96, 96), pl.ds(kw * 128, 128)]
    u1_ref[pl.ds(288, 8), :] = jnp.ones((8, G * 128), bf)
    z1 = jnp.dot(g1_ref[...], u1_ref[...], preferred_element_type=f32)
    a1_ref[...] = _gelu_erf(z1).astype(bf)

    y2_ref[...] = jnp.dot(g2s_ref[...], a1_ref[...],
                          preferred_element_type=f32).astype(bf)
    for g in range(G):
        for kw in range(9):
            yc_ref[pl.ds(g * 32, 32), pl.ds(kw * 128, 128)] = \
                y2_ref[pl.ds(kw * 32, 32), pl.ds(g * 128, 128)]
    z2 = jnp.dot(yc_ref[...], c2x_ref[...],
                 preferred_element_type=f32) + b2_ref[...]
    a2_ref[...] = _gelu_erf(z2).astype(bf)

    p3_ref[...] = jnp.dot(bd_ref[...], a2_ref[...],
                          preferred_element_type=f32).astype(bf)
    acc = jnp.dot(p3_ref[pl.ds(0, 128), :], c3v_ref[pl.ds(0, 32), :],
                  preferred_element_type=f32)
    acc = acc + jnp.dot(p3_ref[pl.ds(128, 128), :], c3v_ref[pl.ds(32, 32), :],
                        preferred_element_type=f32)
    acc = acc + jnp.dot(p3_ref[pl.ds(256, 128), :], c3v_ref[pl.ds(64, 32), :],
                        preferred_element_type=f32)
    o_ref[...] = acc + k3r_ref[...]


def kernel(x, g1a, c1b, g2a, c2b, g3c, c3b, k3):
    f32 = jnp.float32
    bf = jnp.bfloat16
    G, B = _G, _B

    n = x.shape[0]
    nb = -(-n // B)
    nsteps = max(-(-nb // G), 2)
    nbp = nsteps * G
    if nbp * B != n:
        x = jnp.pad(x, ((0, nbp * B - n), (0, 0), (0, 0), (0, 0)))

    xb = (x.reshape(nbp, B, 96, 32)
           .transpose(0, 2, 1, 3)
           .reshape(nsteps, G * 96, B * 32)
           .astype(bf))

    c1h = jnp.concatenate([c1b[0], c1b[1], c1b[2]], axis=1).astype(bf)
    g1b = g1a.astype(bf)
    g2s = (g2a[:, :1080].reshape(32, 9, 120)
           .transpose(1, 0, 2).reshape(288, 120).astype(bf))
    c2x = jnp.concatenate([c2b[kw] for kw in range(9)], axis=0).astype(bf)
    b2r = jnp.tile(g2a[:, 1080:1088].sum(axis=1)[:, None], (G, 32))
    eye_g = jnp.eye(G, dtype=f32)
    bdall = jnp.concatenate(
        [jnp.kron(eye_g, g3c[:, kw * 32:(kw + 1) * 32]) for kw in range(3)],
        axis=0).astype(bf)
    c3v = jnp.concatenate([c3b[0], c3b[1], c3b[2]], axis=0).astype(bf)
    k3r = jnp.tile(k3, (G, 1))

    def batch_spec(shape):
        return pl.BlockSpec((None,) + tuple(shape),
                            lambda s: (s,) + (0,) * len(shape))

    def const_spec(a):
        return pl.BlockSpec(a.shape, lambda s: (0,) * a.ndim)

    out = pl.pallas_call(
        _fused_kernel,
        out_shape=jax.ShapeDtypeStruct((nsteps, G * 16, 32), f32),
        grid=(nsteps,),
        in_specs=[
            batch_spec((G * 96, B * 32)),
            const_spec(c1h), const_spec(g1b), const_spec(g2s),
            const_spec(c2x), const_spec(b2r), const_spec(bdall),
            const_spec(c3v), const_spec(k3r),
        ],
        out_specs=batch_spec((G * 16, 32)),
        scratch_shapes=[
            pltpu.VMEM((G * 96, 384), bf),
            pltpu.VMEM((296, G * 128), bf),
            pltpu.VMEM((120, G * 128), bf),
            pltpu.VMEM((288, G * 128), bf),
            pltpu.VMEM((G * 32, 1152), bf),
            pltpu.VMEM((G * 32, 32), bf),
            pltpu.VMEM((384, 32), bf),
        ],
        compiler_params=pltpu.CompilerParams(
            dimension_semantics=("parallel",)),
    )(xb, c1h, g1b, g2s, c2x, b2r, bdall, c3v, k3r)

    out = (out.reshape(nbp, 2, 8, B, 8)
              .transpose(0, 3, 1, 2, 4)
              .reshape(nbp * B, 2, 8, 8))
    return out[:n]


# speedup vs baseline: 4.0647x; 2.2179x over previous
import jax
import jax.numpy as jnp
from jax.experimental import pallas as pl
from jax.experimental.pallas import tpu as pltpu

_INV_SQRT2 = 0.7071067811865476
_G = 8
_B = 4


def _gelu_erf(v):
    return (v * 0.5) * (jax.lax.erf(v * _INV_SQRT2) + 1.0)


def _fused_kernel(x_ref, c1h_ref, g1_ref, g2s_ref, c2x_ref, b2_ref,
                  bd_ref, c3v_ref, k3r_ref, o_ref,
                  xr_ref, t1_ref, u1_ref, a1_ref, y2_ref, yc_ref, a2_ref,
                  p3_ref):
    f32 = jnp.float32
    bf = jnp.bfloat16
    G = _G

    xv = x_ref[...].astype(bf)
    xr_ref[...] = (xv.reshape(G, 4, 96, 32)
                     .transpose(0, 2, 1, 3)
                     .reshape(G * 96, 128))

    t1_ref[...] = jnp.dot(xr_ref[...], c1h_ref[...],
                          preferred_element_type=f32).astype(bf)
    for kw in range(3):
        for g in range(G):
            u1_ref[pl.ds(kw * 96, 96), pl.ds(g * 128, 128)] = \
                t1_ref[pl.ds(g * 96, 96), pl.ds(kw * 128, 128)]
    u1_ref[pl.ds(288, 8), :] = jnp.ones((8, G * 128), bf)
    z1 = jnp.dot(g1_ref[...], u1_ref[...], preferred_element_type=f32)
    a1_ref[...] = _gelu_erf(z1).astype(bf)

    y2_ref[...] = jnp.dot(g2s_ref[...], a1_ref[...],
                          preferred_element_type=f32).astype(bf)
    for g in range(G):
        for kw in range(9):
            yc_ref[pl.ds(g * 32, 32), pl.ds(kw * 128, 128)] = \
                y2_ref[pl.ds(kw * 32, 32), pl.ds(g * 128, 128)]
    z2 = jnp.dot(yc_ref[...], c2x_ref[...],
                 preferred_element_type=f32) + b2_ref[...]
    a2_ref[...] = _gelu_erf(z2).astype(bf)

    p3_ref[...] = jnp.dot(bd_ref[...], a2_ref[...],
                          preferred_element_type=f32).astype(bf)
    acc = jnp.dot(p3_ref[pl.ds(0, 128), :], c3v_ref[pl.ds(0, 32), :],
                  preferred_element_type=f32)
    acc = acc + jnp.dot(p3_ref[pl.ds(128, 128), :], c3v_ref[pl.ds(32, 32), :],
                        preferred_element_type=f32)
    acc = acc + jnp.dot(p3_ref[pl.ds(256, 128), :], c3v_ref[pl.ds(64, 32), :],
                        preferred_element_type=f32)
    res = acc + k3r_ref[...]
    o_ref[...] = (res.reshape(G, 16, 4, 8)
                     .transpose(0, 2, 1, 3)
                     .reshape(G * 4, 16, 8))


def kernel(x, g1a, c1b, g2a, c2b, g3c, c3b, k3):
    f32 = jnp.float32
    bf = jnp.bfloat16
    G, B = _G, _B

    n = x.shape[0]
    nb = -(-n // B)
    nsteps = max(-(-nb // G), 2)
    nbp = nsteps * G
    if nbp * B != n:
        x = jnp.pad(x, ((0, nbp * B - n), (0, 0), (0, 0), (0, 0)))

    xb = x.reshape(nsteps, G * B, 96, 32)

    c1h = jnp.concatenate([c1b[0], c1b[1], c1b[2]], axis=1).astype(bf)
    g1b = g1a.astype(bf)
    g2s = (g2a[:, :1080].reshape(32, 9, 120)
           .transpose(1, 0, 2).reshape(288, 120).astype(bf))
    c2x = jnp.concatenate([c2b[kw] for kw in range(9)], axis=0).astype(bf)
    b2r = jnp.tile(g2a[:, 1080:1088].sum(axis=1)[:, None], (G, 32))
    eye_g = jnp.eye(G, dtype=f32)
    bdall = jnp.concatenate(
        [jnp.kron(eye_g, g3c[:, kw * 32:(kw + 1) * 32]) for kw in range(3)],
        axis=0).astype(bf)
    c3v = jnp.concatenate([c3b[0], c3b[1], c3b[2]], axis=0).astype(bf)
    k3r = jnp.tile(k3, (G, 1))

    def batch_spec(shape):
        return pl.BlockSpec((None,) + tuple(shape),
                            lambda s: (s,) + (0,) * len(shape))

    def const_spec(a):
        return pl.BlockSpec(a.shape, lambda s: (0,) * a.ndim)

    out = pl.pallas_call(
        _fused_kernel,
        out_shape=jax.ShapeDtypeStruct((nsteps, G * B, 16, 8), f32),
        grid=(nsteps,),
        in_specs=[
            batch_spec((G * B, 96, 32)),
            const_spec(c1h), const_spec(g1b), const_spec(g2s),
            const_spec(c2x), const_spec(b2r), const_spec(bdall),
            const_spec(c3v), const_spec(k3r),
        ],
        out_specs=batch_spec((G * B, 16, 8)),
        scratch_shapes=[
            pltpu.VMEM((G * 96, 128), bf),
            pltpu.VMEM((G * 96, 384), bf),
            pltpu.VMEM((296, G * 128), bf),
            pltpu.VMEM((120, G * 128), bf),
            pltpu.VMEM((288, G * 128), bf),
            pltpu.VMEM((G * 32, 1152), bf),
            pltpu.VMEM((G * 32, 32), bf),
            pltpu.VMEM((384, 32), bf),
        ],
        compiler_params=pltpu.CompilerParams(
            dimension_semantics=("parallel",)),
    )(xb, c1h, g1b, g2s, c2x, b2r, bdall, c3v, k3r)

    return out.reshape(nbp * B, 2, 8, 8)[:n]
